# Initial kernel scaffold; baseline (speedup 1.0000x reference)
import jax
import jax.numpy as jnp
from jax.experimental import pallas as pl


def kernel(x):
    raise NotImplementedError("write your pallas kernel here")


# speedup vs baseline: 92.5175x; 92.5175x over previous
import jax
import jax.numpy as jnp
from jax.experimental import pallas as pl

_K = 512
_NCOLS = 2048
_ROWS_PER_BLOCK = 256


def _topk_mask_kernel(x_ref, o_ref):
    x = x_ref[...]
    keys = jax.lax.bitcast_convert_type(x, jnp.int32) & jnp.int32(0x7FFFFFFF)
    prefix = jnp.zeros((x.shape[0], 1), jnp.int32)
    for b in range(30, -1, -1):
        cand = prefix | jnp.int32(1 << b)
        cnt = jnp.sum((keys >= cand).astype(jnp.int32), axis=1, keepdims=True)
        prefix = jnp.where(cnt >= _K, cand, prefix)
    cnt = jnp.sum((keys >= prefix).astype(jnp.int32), axis=1, keepdims=True)
    scale = jnp.float32(_NCOLS) / cnt.astype(jnp.float32)
    o_ref[...] = jnp.where(keys >= prefix, x * scale, 0.0)


def kernel(x):
    shape = x.shape
    flat = x.reshape(-1, shape[-1])
    n_rows = flat.shape[0]
    out = pl.pallas_call(
        _topk_mask_kernel,
        grid=(n_rows // _ROWS_PER_BLOCK,),
        in_specs=[pl.BlockSpec((_ROWS_PER_BLOCK, _NCOLS), lambda i: (i, 0))],
        out_specs=pl.BlockSpec((_ROWS_PER_BLOCK, _NCOLS), lambda i: (i, 0)),
        out_shape=jax.ShapeDtypeStruct(flat.shape, flat.dtype),
    )(flat)
    return out.reshape(shape), 0, 0
